# baseline (speedup 1.0000x reference)
import functools

import jax
import jax.numpy as jnp
from jax import lax
from jax.experimental import pallas as pl
from jax.experimental.pallas import tpu as pltpu
from jax.experimental.pallas import tpu_sc as plsc

LVL = 4
BATCH = 2
C = 96
H = 224
W = 224
HW = H * W
NQ = 8192
OUTC = LVL * C

NC = 2
NS = 16
LANES = 16
NW = NC * NS
QPW = NQ // NW
CHUNK = 32
NCHUNK = QPW // CHUNK
IDXC = 4 * CHUNK
NBLK = QPW // LANES

_SPLAT_DNUMS = jax.lax.GatherDimensionNumbers(
    offset_dims=(), collapsed_slice_dims=(0,), start_index_map=(0,))


def _sc_body(tables, xs, ys, out, x_v, y_v, w_v, base_v, idx_v, rows_v,
             out_v, sem):
    wid = lax.axis_index("s") * NC + lax.axis_index("c")
    qbase = wid * QPW
    iota = lax.iota(jnp.int32, LANES)

    for b in range(BATCH):
        pltpu.sync_copy(xs.at[b, pl.ds(qbase, QPW)], x_v)
        pltpu.sync_copy(ys.at[b, pl.ds(qbase, QPW)], y_v)

        def blk(i, _):
            q0 = i * LANES
            xv = x_v[pl.ds(q0, LANES)]
            yv = y_v[pl.ds(q0, LANES)]
            xi = jnp.clip(xv.astype(jnp.int32), 0, W - 2)
            yi = jnp.clip(yv.astype(jnp.int32), 0, H - 2)
            fx = xv - xi.astype(jnp.float32)
            fy = yv - yi.astype(jnp.float32)
            gx = 1.0 - fx
            gy = 1.0 - fy
            w_v[pl.ds(0 * QPW + q0, LANES)] = gy * gx
            w_v[pl.ds(1 * QPW + q0, LANES)] = gy * fx
            w_v[pl.ds(2 * QPW + q0, LANES)] = fy * gx
            w_v[pl.ds(3 * QPW + q0, LANES)] = fy * fx
            base = yi * W + xi + (b * HW)
            ch = i // 2
            h = i % 2
            d0 = ch * IDXC + h * LANES
            for k, delta in enumerate((0, 1, W, W + 1)):
                base_v[pl.ds(d0 + k * CHUNK, LANES)] = base + delta
            return 0

        lax.fori_loop(0, NBLK, blk, 0)

        def lvl(j, _):
            v = base_v[pl.ds(j * LANES, LANES)]
            for l in range(LVL):
                idx_v[pl.ds(l * (NCHUNK * IDXC) + j * LANES, LANES)] = (
                    v + l * (BATCH * HW))
            return 0

        lax.fori_loop(0, NCHUNK * IDXC // LANES, lvl, 0)

        def round_(ch, _):
            copies = []
            for l in range(LVL):
                idx_ref = idx_v.at[pl.ds(l * (NCHUNK * IDXC) + ch * IDXC,
                                         IDXC)]
                copies.append(pltpu.async_copy(
                    tables.at[idx_ref], rows_v.at[pl.ds(l * IDXC, IDXC)],
                    sem))
            for cp in copies:
                cp.wait()

            def qloop(q, _):
                qb = q // LANES
                qm = lax.broadcast(q % LANES, (LANES,))
                ws = []
                for k in range(4):
                    wv = w_v[pl.ds(k * QPW + ch * CHUNK + qb * LANES, LANES)]
                    ws.append(lax.gather(
                        wv, qm[:, None], _SPLAT_DNUMS, slice_sizes=(1,),
                        mode=lax.GatherScatterMode.PROMISE_IN_BOUNDS))
                for l in range(LVL):
                    for c6 in range(C // LANES):
                        acc = None
                        for k in range(4):
                            g = rows_v[l * IDXC + k * CHUNK + q,
                                       pl.ds(c6 * LANES, LANES)]
                            t = g * ws[k]
                            acc = t if acc is None else acc + t
                        out_v[q, pl.ds(l * C + c6 * LANES, LANES)] = acc
                return 0

            lax.fori_loop(0, CHUNK, qloop, 0)
            pltpu.sync_copy(
                out_v, out.at[b, pl.ds(qbase + ch * CHUNK, CHUNK)])
            return 0

        lax.fori_loop(0, NCHUNK, round_, 0)


@jax.jit
def _sc_call(tables, xs, ys):
    mesh = plsc.VectorSubcoreMesh(core_axis_name="c", subcore_axis_name="s")
    return pl.kernel(
        _sc_body,
        out_type=jax.ShapeDtypeStruct((BATCH, NQ, OUTC), jnp.float32),
        mesh=mesh,
        scratch_types=[
            pltpu.VMEM((QPW,), jnp.float32),
            pltpu.VMEM((QPW,), jnp.float32),
            pltpu.VMEM((4 * QPW,), jnp.float32),
            pltpu.VMEM((NCHUNK * IDXC,), jnp.int32),
            pltpu.VMEM((LVL * NCHUNK * IDXC,), jnp.int32),
            pltpu.VMEM((LVL * IDXC, C), jnp.float32),
            pltpu.VMEM((CHUNK, OUTC), jnp.float32),
            pltpu.SemaphoreType.DMA,
        ],
        compiler_params=pltpu.CompilerParams(use_tc_tiling_on_sc=False),
    )(tables, xs, ys)


def kernel(input_feats, input_coords, input_size):
    tables = input_feats.transpose(0, 1, 3, 4, 2).reshape(LVL * BATCH * HW, C)
    xs = input_coords[:, :, 0] * ((W - 1.0) / input_size)
    ys = input_coords[:, :, 1] * ((H - 1.0) / input_size)
    out = _sc_call(tables, xs, ys)
    return (out[0], out[1])


# speedup vs baseline: 1.6772x; 1.6772x over previous
import functools

import jax
import jax.numpy as jnp
from jax import lax
from jax.experimental import pallas as pl
from jax.experimental.pallas import tpu as pltpu
from jax.experimental.pallas import tpu_sc as plsc

LVL = 4
BATCH = 2
LB = LVL * BATCH
C = 96
CPAD = 128
H = 224
W = 224
HW = H * W
NQ = 8192
OUTC = LVL * C

NC = 2
NS = 16
LANES = 16
NW = NC * NS
QPW = NQ // NW
CHUNK = 32
NCHUNK = QPW // CHUNK
IDXC = 4 * CHUNK
NBLK = QPW // LANES

TBLK = 1024
NTBLK = HW // TBLK

_SPLAT_DNUMS = jax.lax.GatherDimensionNumbers(
    offset_dims=(), collapsed_slice_dims=(0,), start_index_map=(0,))


def _tr_body(x_ref, o_ref):
    o_ref[:, :C] = x_ref[0].T


@jax.jit
def _build_tables(feats3):
    return pl.pallas_call(
        _tr_body,
        out_shape=jax.ShapeDtypeStruct((LB * HW, CPAD), jnp.float32),
        grid=(LB, NTBLK),
        in_specs=[pl.BlockSpec((1, C, TBLK), lambda i, j: (i, 0, j))],
        out_specs=pl.BlockSpec((TBLK, CPAD), lambda i, j: (i * NTBLK + j, 0)),
    )(feats3)


def _sc_body(tables, xs, ys, out, x_v, y_v, w_v, base_v, idx_v, rows_v,
             out_v, sem):
    wid = lax.axis_index("s") * NC + lax.axis_index("c")
    qbase = wid * QPW
    iota = lax.iota(jnp.int32, LANES)

    for b in range(BATCH):
        pltpu.sync_copy(xs.at[pl.ds(b * NQ + qbase, QPW)], x_v)
        pltpu.sync_copy(ys.at[pl.ds(b * NQ + qbase, QPW)], y_v)

        def blk(i, _):
            q0 = i * LANES
            xv = x_v[pl.ds(q0, LANES)]
            yv = y_v[pl.ds(q0, LANES)]
            xi = jnp.clip(xv.astype(jnp.int32), 0, W - 2)
            yi = jnp.clip(yv.astype(jnp.int32), 0, H - 2)
            fx = xv - xi.astype(jnp.float32)
            fy = yv - yi.astype(jnp.float32)
            gx = 1.0 - fx
            gy = 1.0 - fy
            w_v[pl.ds(0 * QPW + q0, LANES)] = gy * gx
            w_v[pl.ds(1 * QPW + q0, LANES)] = gy * fx
            w_v[pl.ds(2 * QPW + q0, LANES)] = fy * gx
            w_v[pl.ds(3 * QPW + q0, LANES)] = fy * fx
            base = yi * W + xi + (b * HW)
            ch = i // 2
            h = i % 2
            d0 = ch * IDXC + h * LANES
            for k, delta in enumerate((0, 1, W, W + 1)):
                base_v[pl.ds(d0 + k * CHUNK, LANES)] = base + delta
            return 0

        lax.fori_loop(0, NBLK, blk, 0)

        def lvl(j, _):
            v = base_v[pl.ds(j * LANES, LANES)]
            for l in range(LVL):
                idx_v[pl.ds(l * (NCHUNK * IDXC) + j * LANES, LANES)] = (
                    v + l * (BATCH * HW))
            return 0

        lax.fori_loop(0, NCHUNK * IDXC // LANES, lvl, 0)

        def round_(ch, _):
            copies = []
            for l in range(LVL):
                idx_ref = idx_v.at[pl.ds(l * (NCHUNK * IDXC) + ch * IDXC,
                                         IDXC)]
                copies.append(pltpu.async_copy(
                    tables.at[idx_ref], rows_v.at[pl.ds(l * IDXC, IDXC)],
                    sem))
            for cp in copies:
                cp.wait()

            def qloop(q, _):
                qb = q // LANES
                qm = lax.broadcast(q % LANES, (LANES,))
                ws = []
                for k in range(4):
                    wv = w_v[pl.ds(k * QPW + ch * CHUNK + qb * LANES, LANES)]
                    ws.append(lax.gather(
                        wv, qm[:, None], _SPLAT_DNUMS, slice_sizes=(1,),
                        mode=lax.GatherScatterMode.PROMISE_IN_BOUNDS))
                for l in range(LVL):
                    for c6 in range(C // LANES):
                        acc = None
                        for k in range(4):
                            g = rows_v[l * IDXC + k * CHUNK + q,
                                       pl.ds(c6 * LANES, LANES)]
                            t = g * ws[k]
                            acc = t if acc is None else acc + t
                        out_v[q, pl.ds(l * C + c6 * LANES, LANES)] = acc
                return 0

            lax.fori_loop(0, CHUNK, qloop, 0)
            pltpu.sync_copy(
                out_v, out.at[b, pl.ds(qbase + ch * CHUNK, CHUNK)])
            return 0

        lax.fori_loop(0, NCHUNK, round_, 0)


@jax.jit
def _sc_call(tables, xs, ys):
    mesh = plsc.VectorSubcoreMesh(core_axis_name="c", subcore_axis_name="s")
    return pl.kernel(
        _sc_body,
        out_type=jax.ShapeDtypeStruct((BATCH, NQ, OUTC), jnp.float32),
        mesh=mesh,
        scratch_types=[
            pltpu.VMEM((QPW,), jnp.float32),
            pltpu.VMEM((QPW,), jnp.float32),
            pltpu.VMEM((4 * QPW,), jnp.float32),
            pltpu.VMEM((NCHUNK * IDXC,), jnp.int32),
            pltpu.VMEM((LVL * NCHUNK * IDXC,), jnp.int32),
            pltpu.VMEM((LVL * IDXC, CPAD), jnp.float32),
            pltpu.VMEM((CHUNK, OUTC), jnp.float32),
            pltpu.SemaphoreType.DMA,
        ],
    )(tables, xs, ys)


def kernel(input_feats, input_coords, input_size):
    feats3 = input_feats.reshape(LB, C, HW)
    tables = _build_tables(feats3)
    xs = (input_coords[:, :, 0] * ((W - 1.0) / input_size)).reshape(-1)
    ys = (input_coords[:, :, 1] * ((H - 1.0) / input_size)).reshape(-1)
    out = _sc_call(tables, xs, ys)
    return (out[0], out[1])


# speedup vs baseline: 2.1068x; 1.2561x over previous
import functools

import jax
import jax.numpy as jnp
from jax import lax
from jax.experimental import pallas as pl
from jax.experimental.pallas import tpu as pltpu
from jax.experimental.pallas import tpu_sc as plsc

LVL = 4
BATCH = 2
LB = LVL * BATCH
C = 96
CPAD = 128
H = 224
W = 224
HW = H * W
NQ = 8192
OUTC = LVL * C

NC = 2
NS = 16
LANES = 16
NW = NC * NS
QPW = NQ // NW
CHUNK = 32
NCHUNK = QPW // CHUNK
IDXC = 4 * CHUNK
NBLK = QPW // LANES

TBLK = 3584
NTBLK = HW // TBLK

_SPLAT_DNUMS = jax.lax.GatherDimensionNumbers(
    offset_dims=(), collapsed_slice_dims=(0,), start_index_map=(0,))


def _tr_body(x_ref, o_ref):
    eye = (lax.broadcasted_iota(jnp.int32, (C, CPAD), 0)
           == lax.broadcasted_iota(jnp.int32, (C, CPAD), 1)
           ).astype(jnp.float32)
    o_ref[...] = lax.dot_general(
        x_ref[0], eye, (((0,), (0,)), ((), ())),
        preferred_element_type=jnp.float32)


@jax.jit
def _build_tables(feats3):
    return pl.pallas_call(
        _tr_body,
        out_shape=jax.ShapeDtypeStruct((LB * HW, CPAD), jnp.float32),
        grid=(LB, NTBLK),
        in_specs=[pl.BlockSpec((1, C, TBLK), lambda i, j: (i, 0, j))],
        out_specs=pl.BlockSpec((TBLK, CPAD), lambda i, j: (i * NTBLK + j, 0)),
    )(feats3)


def _sc_body(tables, xs, ys, out, x_v, y_v, w_v, base_v, idx_v, rows_v,
             out_v, sem):
    wid = lax.axis_index("s") * NC + lax.axis_index("c")
    qbase = wid * QPW
    iota = lax.iota(jnp.int32, LANES)

    for b in range(BATCH):
        pltpu.sync_copy(xs.at[pl.ds(b * NQ + qbase, QPW)], x_v)
        pltpu.sync_copy(ys.at[pl.ds(b * NQ + qbase, QPW)], y_v)

        def blk(i, _):
            q0 = i * LANES
            xv = x_v[pl.ds(q0, LANES)]
            yv = y_v[pl.ds(q0, LANES)]
            xi = jnp.clip(xv.astype(jnp.int32), 0, W - 2)
            yi = jnp.clip(yv.astype(jnp.int32), 0, H - 2)
            fx = xv - xi.astype(jnp.float32)
            fy = yv - yi.astype(jnp.float32)
            gx = 1.0 - fx
            gy = 1.0 - fy
            w_v[pl.ds(0 * QPW + q0, LANES)] = gy * gx
            w_v[pl.ds(1 * QPW + q0, LANES)] = gy * fx
            w_v[pl.ds(2 * QPW + q0, LANES)] = fy * gx
            w_v[pl.ds(3 * QPW + q0, LANES)] = fy * fx
            base = yi * W + xi + (b * HW)
            ch = i // 2
            h = i % 2
            d0 = ch * IDXC + h * LANES
            for k, delta in enumerate((0, 1, W, W + 1)):
                base_v[pl.ds(d0 + k * CHUNK, LANES)] = base + delta
            return 0

        lax.fori_loop(0, NBLK, blk, 0)

        def lvl(j, _):
            v = base_v[pl.ds(j * LANES, LANES)]
            for l in range(LVL):
                idx_v[pl.ds(l * (NCHUNK * IDXC) + j * LANES, LANES)] = (
                    v + l * (BATCH * HW))
            return 0

        lax.fori_loop(0, NCHUNK * IDXC // LANES, lvl, 0)

        def round_(ch, _):
            copies = []
            for l in range(LVL):
                idx_ref = idx_v.at[pl.ds(l * (NCHUNK * IDXC) + ch * IDXC,
                                         IDXC)]
                copies.append(pltpu.async_copy(
                    tables.at[idx_ref], rows_v.at[pl.ds(l * IDXC, IDXC)],
                    sem))
            for cp in copies:
                cp.wait()

            def qloop(q, _):
                qb = q // LANES
                qm = lax.broadcast(q % LANES, (LANES,))
                ws = []
                for k in range(4):
                    wv = w_v[pl.ds(k * QPW + ch * CHUNK + qb * LANES, LANES)]
                    ws.append(lax.gather(
                        wv, qm[:, None], _SPLAT_DNUMS, slice_sizes=(1,),
                        mode=lax.GatherScatterMode.PROMISE_IN_BOUNDS))
                for l in range(LVL):
                    for c6 in range(C // LANES):
                        acc = None
                        for k in range(4):
                            g = rows_v[l * IDXC + k * CHUNK + q,
                                       pl.ds(c6 * LANES, LANES)]
                            t = g * ws[k]
                            acc = t if acc is None else acc + t
                        out_v[q, pl.ds(l * C + c6 * LANES, LANES)] = acc
                return 0

            lax.fori_loop(0, CHUNK, qloop, 0)
            pltpu.sync_copy(
                out_v, out.at[b, pl.ds(qbase + ch * CHUNK, CHUNK)])
            return 0

        lax.fori_loop(0, NCHUNK, round_, 0)


@jax.jit
def _sc_call(tables, xs, ys):
    mesh = plsc.VectorSubcoreMesh(core_axis_name="c", subcore_axis_name="s")
    return pl.kernel(
        _sc_body,
        out_type=jax.ShapeDtypeStruct((BATCH, NQ, OUTC), jnp.float32),
        mesh=mesh,
        scratch_types=[
            pltpu.VMEM((QPW,), jnp.float32),
            pltpu.VMEM((QPW,), jnp.float32),
            pltpu.VMEM((4 * QPW,), jnp.float32),
            pltpu.VMEM((NCHUNK * IDXC,), jnp.int32),
            pltpu.VMEM((LVL * NCHUNK * IDXC,), jnp.int32),
            pltpu.VMEM((LVL * IDXC, CPAD), jnp.float32),
            pltpu.VMEM((CHUNK, OUTC), jnp.float32),
            pltpu.SemaphoreType.DMA,
        ],
    )(tables, xs, ys)


def kernel(input_feats, input_coords, input_size):
    feats3 = input_feats.reshape(LB, C, HW)
    tables = _build_tables(feats3)
    xs = (input_coords[:, :, 0] * ((W - 1.0) / input_size)).reshape(-1)
    ys = (input_coords[:, :, 1] * ((H - 1.0) / input_size)).reshape(-1)
    out = _sc_call(tables, xs, ys)
    return (out[0], out[1])


# speedup vs baseline: 2.3024x; 1.0928x over previous
import functools

import jax
import jax.numpy as jnp
from jax import lax
from jax.experimental import pallas as pl
from jax.experimental.pallas import tpu as pltpu
from jax.experimental.pallas import tpu_sc as plsc

LVL = 4
BATCH = 2
LB = LVL * BATCH
C = 96
CPAD = 128
H = 224
W = 224
HW = H * W
NQ = 8192
OUTC = LVL * C

NC = 2
NS = 16
LANES = 16
NW = NC * NS
QPW = NQ // NW
CHUNK = 32
NCHUNK = QPW // CHUNK
IDXC = 4 * CHUNK
NBLK = QPW // LANES

TBLK = 3584
NTBLK = HW // TBLK

_SPLAT_DNUMS = jax.lax.GatherDimensionNumbers(
    offset_dims=(), collapsed_slice_dims=(0,), start_index_map=(0,))


def _tr_body(x_ref, o_ref):
    eye = (lax.broadcasted_iota(jnp.int32, (C, CPAD), 0)
           == lax.broadcasted_iota(jnp.int32, (C, CPAD), 1)
           ).astype(jnp.float32)
    o_ref[...] = lax.dot_general(
        x_ref[0], eye, (((0,), (0,)), ((), ())),
        preferred_element_type=jnp.float32)


@jax.jit
def _build_tables(feats3):
    return pl.pallas_call(
        _tr_body,
        out_shape=jax.ShapeDtypeStruct((LB * HW, CPAD), jnp.float32),
        grid=(LB, NTBLK),
        in_specs=[pl.BlockSpec((1, C, TBLK), lambda i, j: (i, 0, j))],
        out_specs=pl.BlockSpec((TBLK, CPAD), lambda i, j: (i * NTBLK + j, 0)),
    )(feats3)


def _sc_body(tables, xs, ys, out, x_v, y_v, w_v, base_v, idx_v, rows_v,
             out_v, *sems):
    wid = lax.axis_index("s") * NC + lax.axis_index("c")
    qbase = wid * QPW
    iota = lax.iota(jnp.int32, LANES)

    for b in range(BATCH):
        pltpu.sync_copy(xs.at[pl.ds(b * NQ + qbase, QPW)], x_v)
        pltpu.sync_copy(ys.at[pl.ds(b * NQ + qbase, QPW)], y_v)

        def blk(i, _):
            q0 = i * LANES
            xv = x_v[pl.ds(q0, LANES)]
            yv = y_v[pl.ds(q0, LANES)]
            xi = jnp.clip(xv.astype(jnp.int32), 0, W - 2)
            yi = jnp.clip(yv.astype(jnp.int32), 0, H - 2)
            fx = xv - xi.astype(jnp.float32)
            fy = yv - yi.astype(jnp.float32)
            gx = 1.0 - fx
            gy = 1.0 - fy
            w_v[pl.ds(0 * QPW + q0, LANES)] = gy * gx
            w_v[pl.ds(1 * QPW + q0, LANES)] = gy * fx
            w_v[pl.ds(2 * QPW + q0, LANES)] = fy * gx
            w_v[pl.ds(3 * QPW + q0, LANES)] = fy * fx
            base = yi * W + xi + (b * HW)
            ch = i // 2
            h = i % 2
            d0 = ch * IDXC + h * LANES
            for k, delta in enumerate((0, 1, W, W + 1)):
                base_v[pl.ds(d0 + k * CHUNK, LANES)] = base + delta
            return 0

        lax.fori_loop(0, NBLK, blk, 0)

        def lvl(j, _):
            v = base_v[pl.ds(j * LANES, LANES)]
            for l in range(LVL):
                idx_v[pl.ds(l * (NCHUNK * IDXC) + j * LANES, LANES)] = (
                    v + l * (BATCH * HW))
            return 0

        lax.fori_loop(0, NCHUNK * IDXC // LANES, lvl, 0)

        def issue(ch, l):
            idx_ref = idx_v.at[pl.ds(l * (NCHUNK * IDXC) + ch * IDXC, IDXC)]
            return pltpu.async_copy(
                tables.at[idx_ref], rows_v.at[pl.ds(l * IDXC, IDXC)],
                sems[l])

        for l in range(LVL):
            issue(0, l)

        def round_(ch, _):
            for l in range(LVL):
                pltpu.make_async_copy(
                    tables.at[idx_v.at[pl.ds(0, IDXC)]],
                    rows_v.at[pl.ds(l * IDXC, IDXC)], sems[l]).wait()

                def qloop(q, _):
                    qb = q // LANES
                    qm = lax.broadcast(q % LANES, (LANES,))
                    ws = []
                    for k in range(4):
                        wv = w_v[pl.ds(k * QPW + ch * CHUNK + qb * LANES,
                                       LANES)]
                        ws.append(lax.gather(
                            wv, qm[:, None], _SPLAT_DNUMS, slice_sizes=(1,),
                            mode=lax.GatherScatterMode.PROMISE_IN_BOUNDS))
                    for c6 in range(C // LANES):
                        acc = None
                        for k in range(4):
                            g = rows_v[l * IDXC + k * CHUNK + q,
                                       pl.ds(c6 * LANES, LANES)]
                            t = g * ws[k]
                            acc = t if acc is None else acc + t
                        out_v[q, pl.ds(l * C + c6 * LANES, LANES)] = acc
                    return 0

                lax.fori_loop(0, CHUNK, qloop, 0)

                @pl.when(ch + 1 < NCHUNK)
                def _():
                    issue(ch + 1, l)

            pltpu.sync_copy(
                out_v, out.at[b, pl.ds(qbase + ch * CHUNK, CHUNK)])
            return 0

        lax.fori_loop(0, NCHUNK, round_, 0)


@jax.jit
def _sc_call(tables, xs, ys):
    mesh = plsc.VectorSubcoreMesh(core_axis_name="c", subcore_axis_name="s")
    return pl.kernel(
        _sc_body,
        out_type=jax.ShapeDtypeStruct((BATCH, NQ, OUTC), jnp.float32),
        mesh=mesh,
        scratch_types=[
            pltpu.VMEM((QPW,), jnp.float32),
            pltpu.VMEM((QPW,), jnp.float32),
            pltpu.VMEM((4 * QPW,), jnp.float32),
            pltpu.VMEM((NCHUNK * IDXC,), jnp.int32),
            pltpu.VMEM((LVL * NCHUNK * IDXC,), jnp.int32),
            pltpu.VMEM((LVL * IDXC, CPAD), jnp.float32),
            pltpu.VMEM((CHUNK, OUTC), jnp.float32),
            pltpu.SemaphoreType.DMA,
            pltpu.SemaphoreType.DMA,
            pltpu.SemaphoreType.DMA,
            pltpu.SemaphoreType.DMA,
        ],
    )(tables, xs, ys)


def kernel(input_feats, input_coords, input_size):
    feats3 = input_feats.reshape(LB, C, HW)
    tables = _build_tables(feats3)
    xs = (input_coords[:, :, 0] * ((W - 1.0) / input_size)).reshape(-1)
    ys = (input_coords[:, :, 1] * ((H - 1.0) / input_size)).reshape(-1)
    out = _sc_call(tables, xs, ys)
    return (out[0], out[1])


# speedup vs baseline: 2.5216x; 1.0952x over previous
import functools

import jax
import jax.numpy as jnp
from jax import lax
from jax.experimental import pallas as pl
from jax.experimental.pallas import tpu as pltpu
from jax.experimental.pallas import tpu_sc as plsc

LVL = 4
BATCH = 2
LB = LVL * BATCH
C = 96
CPAD = 128
H = 224
W = 224
HW = H * W
NQ = 8192
OUTC = LVL * C

NC = 2
NS = 16
LANES = 16
NW = NC * NS
QPW = NQ // NW
CHUNK = 32
NCHUNK = QPW // CHUNK
IDXC = 4 * CHUNK
NBLK = QPW // LANES

TBLK = 12544
NTBLK = HW // TBLK

_SPLAT_DNUMS = jax.lax.GatherDimensionNumbers(
    offset_dims=(), collapsed_slice_dims=(0,), start_index_map=(0,))


def _tr_body(x_ref, o_ref):
    eye = (lax.broadcasted_iota(jnp.int32, (C, CPAD), 0)
           == lax.broadcasted_iota(jnp.int32, (C, CPAD), 1)
           ).astype(jnp.float32)
    o_ref[...] = lax.dot_general(
        x_ref[0], eye, (((0,), (0,)), ((), ())),
        preferred_element_type=jnp.float32)


@jax.jit
def _build_tables(feats3):
    return pl.pallas_call(
        _tr_body,
        out_shape=jax.ShapeDtypeStruct((LB * HW, CPAD), jnp.float32),
        grid=(LB, NTBLK),
        in_specs=[pl.BlockSpec((1, C, TBLK), lambda i, j: (i, 0, j))],
        out_specs=pl.BlockSpec((TBLK, CPAD), lambda i, j: (i * NTBLK + j, 0)),
    )(feats3)


def _sc_body(tables, xs, ys, out, x_v, y_v, w_v, base_v, idx_v, rows_v,
             out_v, *sems):
    wid = lax.axis_index("s") * NC + lax.axis_index("c")
    qbase = wid * QPW
    iota = lax.iota(jnp.int32, LANES)

    for b in range(BATCH):
        pltpu.sync_copy(xs.at[pl.ds(b * NQ + qbase, QPW)], x_v)
        pltpu.sync_copy(ys.at[pl.ds(b * NQ + qbase, QPW)], y_v)

        def blk(i, _):
            q0 = i * LANES
            xv = x_v[pl.ds(q0, LANES)]
            yv = y_v[pl.ds(q0, LANES)]
            xi = jnp.clip(xv.astype(jnp.int32), 0, W - 2)
            yi = jnp.clip(yv.astype(jnp.int32), 0, H - 2)
            fx = xv - xi.astype(jnp.float32)
            fy = yv - yi.astype(jnp.float32)
            gx = 1.0 - fx
            gy = 1.0 - fy
            w_v[pl.ds(0 * QPW + q0, LANES)] = gy * gx
            w_v[pl.ds(1 * QPW + q0, LANES)] = gy * fx
            w_v[pl.ds(2 * QPW + q0, LANES)] = fy * gx
            w_v[pl.ds(3 * QPW + q0, LANES)] = fy * fx
            base = yi * W + xi + (b * HW)
            ch = i // 2
            h = i % 2
            d0 = ch * IDXC + h * LANES
            for k, delta in enumerate((0, 1, W, W + 1)):
                base_v[pl.ds(d0 + k * CHUNK, LANES)] = base + delta
            return 0

        lax.fori_loop(0, NBLK, blk, 0)

        def lvl(j, _):
            v = base_v[pl.ds(j * LANES, LANES)]
            for l in range(LVL):
                idx_v[pl.ds(l * (NCHUNK * IDXC) + j * LANES, LANES)] = (
                    v + l * (BATCH * HW))
            return 0

        lax.fori_loop(0, NCHUNK * IDXC // LANES, lvl, 0)

        def issue(ch, l):
            idx_ref = idx_v.at[pl.ds(l * (NCHUNK * IDXC) + ch * IDXC, IDXC)]
            return pltpu.async_copy(
                tables.at[idx_ref], rows_v.at[pl.ds(l * IDXC, IDXC)],
                sems[l])

        for l in range(LVL):
            issue(0, l)

        def round_(ch, _):
            for l in range(LVL):
                pltpu.make_async_copy(
                    tables.at[idx_v.at[pl.ds(0, IDXC)]],
                    rows_v.at[pl.ds(l * IDXC, IDXC)], sems[l]).wait()

                def qloop(q, _):
                    qb = q // LANES
                    qm = lax.broadcast(q % LANES, (LANES,))
                    ws = []
                    for k in range(4):
                        wv = w_v[pl.ds(k * QPW + ch * CHUNK + qb * LANES,
                                       LANES)]
                        ws.append(lax.gather(
                            wv, qm[:, None], _SPLAT_DNUMS, slice_sizes=(1,),
                            mode=lax.GatherScatterMode.PROMISE_IN_BOUNDS))
                    for c6 in range(C // LANES):
                        acc = None
                        for k in range(4):
                            g = rows_v[l * IDXC + k * CHUNK + q,
                                       pl.ds(c6 * LANES, LANES)]
                            t = g * ws[k]
                            acc = t if acc is None else acc + t
                        out_v[q, pl.ds(l * C + c6 * LANES, LANES)] = acc
                    return 0

                lax.fori_loop(0, CHUNK, qloop, 0)

                @pl.when(ch + 1 < NCHUNK)
                def _():
                    issue(ch + 1, l)

            pltpu.sync_copy(
                out_v, out.at[b, pl.ds(qbase + ch * CHUNK, CHUNK)])
            return 0

        lax.fori_loop(0, NCHUNK, round_, 0)


@jax.jit
def _sc_call(tables, xs, ys):
    mesh = plsc.VectorSubcoreMesh(core_axis_name="c", subcore_axis_name="s")
    return pl.kernel(
        _sc_body,
        out_type=jax.ShapeDtypeStruct((BATCH, NQ, OUTC), jnp.float32),
        mesh=mesh,
        scratch_types=[
            pltpu.VMEM((QPW,), jnp.float32),
            pltpu.VMEM((QPW,), jnp.float32),
            pltpu.VMEM((4 * QPW,), jnp.float32),
            pltpu.VMEM((NCHUNK * IDXC,), jnp.int32),
            pltpu.VMEM((LVL * NCHUNK * IDXC,), jnp.int32),
            pltpu.VMEM((LVL * IDXC, CPAD), jnp.float32),
            pltpu.VMEM((CHUNK, OUTC), jnp.float32),
            pltpu.SemaphoreType.DMA,
            pltpu.SemaphoreType.DMA,
            pltpu.SemaphoreType.DMA,
            pltpu.SemaphoreType.DMA,
        ],
    )(tables, xs, ys)


def kernel(input_feats, input_coords, input_size):
    feats3 = input_feats.reshape(LB, C, HW)
    tables = _build_tables(feats3)
    xs = (input_coords[:, :, 0] * ((W - 1.0) / input_size)).reshape(-1)
    ys = (input_coords[:, :, 1] * ((H - 1.0) / input_size)).reshape(-1)
    out = _sc_call(tables, xs, ys)
    return (out[0], out[1])


# speedup vs baseline: 2.5344x; 1.0051x over previous
import functools

import jax
import jax.numpy as jnp
from jax import lax
from jax.experimental import pallas as pl
from jax.experimental.pallas import tpu as pltpu
from jax.experimental.pallas import tpu_sc as plsc

LVL = 4
BATCH = 2
LB = LVL * BATCH
C = 96
CPAD = 128
H = 224
W = 224
HW = H * W
NQ = 8192
OUTC = LVL * C

NC = 2
NS = 16
LANES = 16
NW = NC * NS
QPW = NQ // NW
CHUNK = 32
NCHUNK = QPW // CHUNK
IDXC = 4 * CHUNK
NBLK = QPW // LANES

TBLK = 25088
NTBLK = HW // TBLK

_SPLAT_DNUMS = jax.lax.GatherDimensionNumbers(
    offset_dims=(), collapsed_slice_dims=(0,), start_index_map=(0,))


def _tr_body(x_ref, o_ref):
    eye = (lax.broadcasted_iota(jnp.int32, (C, CPAD), 0)
           == lax.broadcasted_iota(jnp.int32, (C, CPAD), 1)
           ).astype(jnp.float32)
    o_ref[...] = lax.dot_general(
        x_ref[0], eye, (((0,), (0,)), ((), ())),
        preferred_element_type=jnp.float32)


@jax.jit
def _build_tables(feats3):
    return pl.pallas_call(
        _tr_body,
        out_shape=jax.ShapeDtypeStruct((LB * HW, CPAD), jnp.float32),
        grid=(LB, NTBLK),
        in_specs=[pl.BlockSpec((1, C, TBLK), lambda i, j: (i, 0, j))],
        out_specs=pl.BlockSpec((TBLK, CPAD), lambda i, j: (i * NTBLK + j, 0)),
    )(feats3)


def _sc_body(tables, xs, ys, out, x_v, y_v, w_v, base_v, idx_v, rows_v,
             out_v, *sems):
    wid = lax.axis_index("s") * NC + lax.axis_index("c")
    qbase = wid * QPW
    iota = lax.iota(jnp.int32, LANES)

    for b in range(BATCH):
        pltpu.sync_copy(xs.at[pl.ds(b * NQ + qbase, QPW)], x_v)
        pltpu.sync_copy(ys.at[pl.ds(b * NQ + qbase, QPW)], y_v)

        def blk(i, _):
            q0 = i * LANES
            xv = x_v[pl.ds(q0, LANES)]
            yv = y_v[pl.ds(q0, LANES)]
            xi = jnp.clip(xv.astype(jnp.int32), 0, W - 2)
            yi = jnp.clip(yv.astype(jnp.int32), 0, H - 2)
            fx = xv - xi.astype(jnp.float32)
            fy = yv - yi.astype(jnp.float32)
            gx = 1.0 - fx
            gy = 1.0 - fy
            w_v[pl.ds(0 * QPW + q0, LANES)] = gy * gx
            w_v[pl.ds(1 * QPW + q0, LANES)] = gy * fx
            w_v[pl.ds(2 * QPW + q0, LANES)] = fy * gx
            w_v[pl.ds(3 * QPW + q0, LANES)] = fy * fx
            base = yi * W + xi + (b * HW)
            ch = i // 2
            h = i % 2
            d0 = ch * IDXC + h * LANES
            for k, delta in enumerate((0, 1, W, W + 1)):
                base_v[pl.ds(d0 + k * CHUNK, LANES)] = base + delta
            return 0

        lax.fori_loop(0, NBLK, blk, 0)

        def lvl(j, _):
            v = base_v[pl.ds(j * LANES, LANES)]
            for l in range(LVL):
                idx_v[pl.ds(l * (NCHUNK * IDXC) + j * LANES, LANES)] = (
                    v + l * (BATCH * HW))
            return 0

        lax.fori_loop(0, NCHUNK * IDXC // LANES, lvl, 0)

        def issue(ch, l):
            idx_ref = idx_v.at[pl.ds(l * (NCHUNK * IDXC) + ch * IDXC, IDXC)]
            return pltpu.async_copy(
                tables.at[idx_ref], rows_v.at[pl.ds(l * IDXC, IDXC)],
                sems[l])

        for l in range(LVL):
            issue(0, l)

        def round_(ch, _):
            for l in range(LVL):
                pltpu.make_async_copy(
                    tables.at[idx_v.at[pl.ds(0, IDXC)]],
                    rows_v.at[pl.ds(l * IDXC, IDXC)], sems[l]).wait()

                def qloop(q, _):
                    qb = q // LANES
                    qm = lax.broadcast(q % LANES, (LANES,))
                    ws = []
                    for k in range(4):
                        wv = w_v[pl.ds(k * QPW + ch * CHUNK + qb * LANES,
                                       LANES)]
                        ws.append(lax.gather(
                            wv, qm[:, None], _SPLAT_DNUMS, slice_sizes=(1,),
                            mode=lax.GatherScatterMode.PROMISE_IN_BOUNDS))
                    for c6 in range(C // LANES):
                        acc = None
                        for k in range(4):
                            g = rows_v[l * IDXC + k * CHUNK + q,
                                       pl.ds(c6 * LANES, LANES)]
                            t = g * ws[k]
                            acc = t if acc is None else acc + t
                        out_v[q, pl.ds(l * C + c6 * LANES, LANES)] = acc
                    return 0

                lax.fori_loop(0, CHUNK, qloop, 0)

                @pl.when(ch + 1 < NCHUNK)
                def _():
                    issue(ch + 1, l)

            pltpu.sync_copy(
                out_v, out.at[b, pl.ds(qbase + ch * CHUNK, CHUNK)])
            return 0

        lax.fori_loop(0, NCHUNK, round_, 0)


@jax.jit
def _sc_call(tables, xs, ys):
    mesh = plsc.VectorSubcoreMesh(core_axis_name="c", subcore_axis_name="s")
    return pl.kernel(
        _sc_body,
        out_type=jax.ShapeDtypeStruct((BATCH, NQ, OUTC), jnp.float32),
        mesh=mesh,
        scratch_types=[
            pltpu.VMEM((QPW,), jnp.float32),
            pltpu.VMEM((QPW,), jnp.float32),
            pltpu.VMEM((4 * QPW,), jnp.float32),
            pltpu.VMEM((NCHUNK * IDXC,), jnp.int32),
            pltpu.VMEM((LVL * NCHUNK * IDXC,), jnp.int32),
            pltpu.VMEM((LVL * IDXC, CPAD), jnp.float32),
            pltpu.VMEM((CHUNK, OUTC), jnp.float32),
            pltpu.SemaphoreType.DMA,
            pltpu.SemaphoreType.DMA,
            pltpu.SemaphoreType.DMA,
            pltpu.SemaphoreType.DMA,
        ],
    )(tables, xs, ys)


def kernel(input_feats, input_coords, input_size):
    feats3 = input_feats.reshape(LB, C, HW)
    tables = _build_tables(feats3)
    xs = (input_coords[:, :, 0] * ((W - 1.0) / input_size)).reshape(-1)
    ys = (input_coords[:, :, 1] * ((H - 1.0) / input_size)).reshape(-1)
    out = _sc_call(tables, xs, ys)
    return (out[0], out[1])
